# baseline (speedup 1.0000x reference)
import functools

import jax
import jax.numpy as jnp
from jax import lax
from jax.experimental import pallas as pl
from jax.experimental.pallas import tpu as pltpu
from jax.experimental.pallas import tpu_sc as plsc

N_USERS = 100000
M_ITEMS = 100000
DIM = 64
BATCH = 4096
K = 8
DECAY = 0.0001

NUM_WORKERS = 32
BPW = BATCH // NUM_WORKERS
LANES = 16

SLABS = N_USERS // 8
RC = 50
N_CHUNKS = SLABS // RC


@functools.cache
def _make_repack_kernel():
  mesh = plsc.VectorSubcoreMesh(core_axis_name="c", subcore_axis_name="s")

  @functools.partial(
      pl.kernel,
      mesh=mesh,
      compiler_params=pltpu.CompilerParams(needs_layout_passes=False,
                                           use_tc_tiling_on_sc=True),
      out_type=[
          jax.ShapeDtypeStruct((N_USERS // 2, 2 * DIM), jnp.float32),
          jax.ShapeDtypeStruct((M_ITEMS // 2, 2 * DIM), jnp.float32),
      ],
      scratch_types=[
          pltpu.VMEM((RC, 8, DIM), jnp.float32),
          pltpu.VMEM((RC * 4, 2 * DIM), jnp.float32),
          pltpu.SemaphoreType.DMA,
      ],
  )
  def _repack(utab_hbm, itab_hbm, upk_hbm, ipk_hbm, a_v, b_v, sem):
    wid = lax.axis_index("s") * 2 + lax.axis_index("c")
    for src, dst in ((utab_hbm, upk_hbm), (itab_hbm, ipk_hbm)):
      src3 = src.reshape(SLABS, 8, DIM)
      for it in range((N_CHUNKS + NUM_WORKERS - 1) // NUM_WORKERS):
        cid = wid + it * NUM_WORKERS

        @pl.when(cid < N_CHUNKS)
        def _(cid=cid, src3=src3, dst=dst):
          pltpu.async_copy(src3.at[pl.ds(cid * RC, RC)], a_v, sem).wait()

          def move(i, carry):
            for r in range(8):
              for c in range(DIM // LANES):
                w = r * DIM + c * LANES
                b_v[i * 4 + w // (2 * DIM),
                    pl.ds(w % (2 * DIM), LANES)] = a_v[i, r,
                                                       pl.ds(c * LANES, LANES)]
            return carry

          lax.fori_loop(0, RC, move, 0)
          pltpu.async_copy(b_v, dst.at[pl.ds(cid * RC * 4, RC * 4)],
                           sem).wait()

  return _repack


@functools.cache
def _make_sc_kernel():
  mesh = plsc.VectorSubcoreMesh(core_axis_name="c", subcore_axis_name="s")

  @functools.partial(
      pl.kernel,
      mesh=mesh,
      compiler_params=pltpu.CompilerParams(needs_layout_passes=False,
                                           use_tc_tiling_on_sc=False),
      out_type=[
          jax.ShapeDtypeStruct((BATCH * K,), jnp.float32),
          jax.ShapeDtypeStruct((NUM_WORKERS, 48), jnp.float32),
      ],
      scratch_types=[
          pltpu.VMEM((BPW,), jnp.int32),
          pltpu.VMEM((BPW,), jnp.int32),
          pltpu.VMEM((K, BPW), jnp.int32),
          pltpu.VMEM((BPW, DIM), jnp.float32),
          pltpu.VMEM((BPW, DIM), jnp.float32),
          pltpu.VMEM((BPW * K, DIM), jnp.float32),
          pltpu.VMEM((BPW * K,), jnp.float32),
          pltpu.VMEM((48,), jnp.float32),
          pltpu.SemaphoreType.DMA,
      ],
  )
  def _sc_gather_score(users_hbm, pos_hbm, neg_hbm, utab_hbm, itab_hbm,
                       pn_hbm, norms_hbm,
                       uidx_v, pidx_v, nidx_v, urows_v, prows_v, nrows_v,
                       pn_v, nrm_v, sem):
    wid = lax.axis_index("s") * 2 + lax.axis_index("c")
    base = wid * BPW

    pltpu.sync_copy(users_hbm.at[pl.ds(base, BPW)], uidx_v)
    pltpu.sync_copy(pos_hbm.at[pl.ds(base, BPW)], pidx_v)
    pltpu.sync_copy(neg_hbm.at[wid], nidx_v)

    copies = [
        pltpu.async_copy(utab_hbm.at[uidx_v], urows_v, sem),
        pltpu.async_copy(itab_hbm.at[pidx_v], prows_v, sem),
    ]
    for j in range(K):
      copies.append(
          pltpu.async_copy(itab_hbm.at[nidx_v.at[j]],
                           nrows_v.at[pl.ds(j * BPW, BPW)], sem))
    for c in copies:
      c.wait()

    zero = jnp.zeros((LANES,), jnp.float32)
    nchunks = DIM // LANES
    iota = lax.iota(jnp.int32, LANES)
    lane15 = iota == 15

    def row_step(b, carry):
      su, sp, sn = carry
      uc = [urows_v[b, pl.ds(c * LANES, LANES)] for c in range(nchunks)]
      pc = [prows_v[b, pl.ds(c * LANES, LANES)] for c in range(nchunks)]
      for c in range(nchunks):
        su = su + uc[c] * uc[c]
        sp = sp + pc[c] * pc[c]
      for k in range(K):
        nb = b * K + k
        nc = [nrows_v[nb, pl.ds(c * LANES, LANES)] for c in range(nchunks)]
        for c in range(nchunks):
          sn = sn + nc[c] * nc[c]
        wd = uc[0] * (pc[0] - nc[0])
        for c in range(1, nchunks):
          wd = wd + uc[c] * (pc[c] - nc[c])
        plsc.store_scatter(pn_v, [iota + (nb - 15)], plsc.cumsum(wd),
                           mask=lane15)
      return su, sp, sn

    s_u, s_p, s_n = lax.fori_loop(0, BPW, row_step, (zero, zero, zero))

    nrm_v[pl.ds(0, LANES)] = s_u
    nrm_v[pl.ds(LANES, LANES)] = s_p
    nrm_v[pl.ds(2 * LANES, LANES)] = s_n * (1.0 / K)

    pltpu.sync_copy(pn_v, pn_hbm.at[pl.ds(base * K, BPW * K)])
    pltpu.sync_copy(nrm_v, norms_hbm.at[wid])

  return _sc_gather_score


def _tc_loss_body(pn_ref, nrm_ref, mf_ref, emb_ref, tot_ref):
  x = -pn_ref[...]
  sp = jnp.maximum(x, 0.0) + jnp.log1p(jnp.exp(-jnp.abs(x)))
  mf = jnp.sum(sp) * (1.0 / (BATCH * K))
  reg = jnp.sum(nrm_ref[...]) * 0.5
  emb = (DECAY / BATCH) * reg
  one = jnp.ones((1, 1), jnp.float32)
  mf_ref[...] = mf * one
  emb_ref[...] = emb * one
  tot_ref[...] = (mf + emb) * one


def kernel(user_table, item_table, users, pos_items, neg_items):
  users_i = users.astype(jnp.int32)
  pos_i = pos_items.astype(jnp.int32)
  neg_i = neg_items.astype(jnp.int32).reshape(NUM_WORKERS, K, BPW)

  upk, ipk = _make_repack_kernel()(user_table, item_table)
  utab_lin = upk.reshape(N_USERS, DIM)
  itab_lin = ipk.reshape(M_ITEMS, DIM)

  pn_flat, norms = _make_sc_kernel()(users_i, pos_i, neg_i,
                                     utab_lin, itab_lin)
  pos_neg = pn_flat.reshape(BATCH, K)

  mf, emb, tot = pl.pallas_call(
      _tc_loss_body,
      out_shape=[jax.ShapeDtypeStruct((1, 1), jnp.float32)] * 3,
  )(pn_flat.reshape(BATCH * K // 128, 128), norms)

  return (tot.reshape(()), mf.reshape(()), emb.reshape(()), pos_neg)


# speedup vs baseline: 1.3301x; 1.3301x over previous
import functools

import jax
import jax.numpy as jnp
from jax import lax
from jax.experimental import pallas as pl
from jax.experimental.pallas import tpu as pltpu
from jax.experimental.pallas import tpu_sc as plsc

N_USERS = 100000
M_ITEMS = 100000
DIM = 64
BATCH = 4096
K = 8
DECAY = 0.0001

NUM_WORKERS = 32
BPW = BATCH // NUM_WORKERS
LANES = 16

@functools.cache
def _make_sc_kernel():
  mesh = plsc.VectorSubcoreMesh(core_axis_name="c", subcore_axis_name="s")

  @functools.partial(
      pl.kernel,
      mesh=mesh,
      compiler_params=pltpu.CompilerParams(needs_layout_passes=False,
                                           use_tc_tiling_on_sc=False),
      out_type=[
          jax.ShapeDtypeStruct((BATCH * K,), jnp.float32),
          jax.ShapeDtypeStruct((NUM_WORKERS, 48), jnp.float32),
      ],
      scratch_types=[
          pltpu.VMEM((BPW,), jnp.int32),
          pltpu.VMEM((BPW,), jnp.int32),
          pltpu.VMEM((K, BPW), jnp.int32),
          pltpu.VMEM((BPW, DIM), jnp.bfloat16),
          pltpu.VMEM((BPW, DIM), jnp.bfloat16),
          pltpu.VMEM((BPW * K, DIM), jnp.bfloat16),
          pltpu.VMEM((BPW * K,), jnp.float32),
          pltpu.VMEM((48,), jnp.float32),
          pltpu.SemaphoreType.DMA,
      ],
  )
  def _sc_gather_score(users_hbm, pos_hbm, neg_hbm, utab_hbm, itab_hbm,
                       pn_hbm, norms_hbm,
                       uidx_v, pidx_v, nidx_v, urows_v, prows_v, nrows_v,
                       pn_v, nrm_v, sem):
    wid = lax.axis_index("s") * 2 + lax.axis_index("c")
    base = wid * BPW

    pltpu.sync_copy(users_hbm.at[pl.ds(base, BPW)], uidx_v)
    pltpu.sync_copy(pos_hbm.at[pl.ds(base, BPW)], pidx_v)
    pltpu.sync_copy(neg_hbm.at[wid], nidx_v)

    copies = [
        pltpu.async_copy(utab_hbm.at[uidx_v], urows_v, sem),
        pltpu.async_copy(itab_hbm.at[pidx_v], prows_v, sem),
    ]
    for j in range(K):
      copies.append(
          pltpu.async_copy(itab_hbm.at[nidx_v.at[j]],
                           nrows_v.at[pl.ds(j * BPW, BPW)], sem))
    for c in copies:
      c.wait()

    zero = jnp.zeros((LANES,), jnp.float32)
    nchunks = DIM // LANES
    iota = lax.iota(jnp.int32, LANES)
    lane15 = iota == 15
    himask = jnp.full((LANES,), -65536, jnp.int32)

    def bf16_row(ref, b):
      out = []
      for c in range(2):
        w = plsc.bitcast(ref[b, pl.ds(c * 2 * LANES, 2 * LANES)], jnp.int32)
        out.append(plsc.bitcast(lax.shift_left(w, 16), jnp.float32))
        out.append(plsc.bitcast(w & himask, jnp.float32))
      return out

    def row_step(b, carry):
      su, sp, sn = carry
      uc = bf16_row(urows_v, b)
      pc = bf16_row(prows_v, b)
      for c in range(nchunks):
        su = su + uc[c] * uc[c]
        sp = sp + pc[c] * pc[c]
      for k in range(K):
        nb = b * K + k
        nc = bf16_row(nrows_v, nb)
        for c in range(nchunks):
          sn = sn + nc[c] * nc[c]
        wd = uc[0] * (pc[0] - nc[0])
        for c in range(1, nchunks):
          wd = wd + uc[c] * (pc[c] - nc[c])
        plsc.store_scatter(pn_v, [iota + (nb - 15)], plsc.cumsum(wd),
                           mask=lane15)
      return su, sp, sn

    s_u, s_p, s_n = lax.fori_loop(0, BPW, row_step, (zero, zero, zero))

    nrm_v[pl.ds(0, LANES)] = s_u
    nrm_v[pl.ds(LANES, LANES)] = s_p
    nrm_v[pl.ds(2 * LANES, LANES)] = s_n * (1.0 / K)

    pltpu.sync_copy(pn_v, pn_hbm.at[pl.ds(base * K, BPW * K)])
    pltpu.sync_copy(nrm_v, norms_hbm.at[wid])

  return _sc_gather_score


def _tc_loss_body(pn_ref, nrm_ref, mf_ref, emb_ref, tot_ref):
  x = -pn_ref[...]
  sp = jnp.maximum(x, 0.0) + jnp.log1p(jnp.exp(-jnp.abs(x)))
  mf = jnp.sum(sp) * (1.0 / (BATCH * K))
  reg = jnp.sum(nrm_ref[...]) * 0.5
  emb = (DECAY / BATCH) * reg
  one = jnp.ones((1, 1), jnp.float32)
  mf_ref[...] = mf * one
  emb_ref[...] = emb * one
  tot_ref[...] = (mf + emb) * one


def kernel(user_table, item_table, users, pos_items, neg_items):
  users_i = users.astype(jnp.int32)
  pos_i = pos_items.astype(jnp.int32)
  neg_i = neg_items.astype(jnp.int32).reshape(NUM_WORKERS, K, BPW)

  utab_bf = user_table.astype(jnp.bfloat16)
  itab_bf = item_table.astype(jnp.bfloat16)

  pn_flat, norms = _make_sc_kernel()(users_i, pos_i, neg_i,
                                     utab_bf, itab_bf)
  pos_neg = pn_flat.reshape(BATCH, K)

  mf, emb, tot = pl.pallas_call(
      _tc_loss_body,
      out_shape=[jax.ShapeDtypeStruct((1, 1), jnp.float32)] * 3,
  )(pn_flat.reshape(BATCH * K // 128, 128), norms)

  return (tot.reshape(()), mf.reshape(()), emb.reshape(()), pos_neg)


# speedup vs baseline: 1.7093x; 1.2851x over previous
import functools

import jax
import jax.numpy as jnp
from jax import lax
from jax.experimental import pallas as pl
from jax.experimental.pallas import tpu as pltpu
from jax.experimental.pallas import tpu_sc as plsc

N_USERS = 100000
M_ITEMS = 100000
DIM = 64
BATCH = 4096
K = 8
DECAY = 0.0001

NUM_WORKERS = 32
BPW = BATCH // NUM_WORKERS
LANES = 16

@functools.cache
def _make_sc_kernel():
  mesh = plsc.VectorSubcoreMesh(core_axis_name="c", subcore_axis_name="s")

  @functools.partial(
      pl.kernel,
      mesh=mesh,
      compiler_params=pltpu.CompilerParams(needs_layout_passes=False,
                                           use_tc_tiling_on_sc=False),
      out_type=[
          jax.ShapeDtypeStruct((BATCH * K,), jnp.float32),
          jax.ShapeDtypeStruct((NUM_WORKERS * 48,), jnp.float32),
      ],
      scratch_types=[
          pltpu.VMEM((BPW,), jnp.int32),
          pltpu.VMEM((BPW,), jnp.int32),
          pltpu.VMEM((K, BPW), jnp.int32),
          pltpu.VMEM((BPW, DIM), jnp.float32),
          pltpu.VMEM((BPW, DIM), jnp.float32),
          pltpu.VMEM((BPW * K, DIM), jnp.float32),
          pltpu.VMEM((BPW * K,), jnp.float32),
          pltpu.VMEM((48,), jnp.float32),
          pltpu.SemaphoreType.DMA,
          [pltpu.SemaphoreType.DMA] * K,
      ],
  )
  def _sc_gather_score(users_hbm, pos_hbm, neg_hbm, utab_hbm, itab_hbm,
                       pn_hbm, norms_hbm,
                       uidx_v, pidx_v, nidx_v, urows_v, prows_v, nrows_v,
                       pn_v, nrm_v, sem, nsems):
    wid = lax.axis_index("s") * 2 + lax.axis_index("c")
    base = wid * BPW

    pltpu.sync_copy(users_hbm.at[pl.ds(base, BPW)], uidx_v)
    pltpu.sync_copy(pos_hbm.at[pl.ds(base, BPW)], pidx_v)
    pltpu.sync_copy(neg_hbm.at[wid], nidx_v)

    cu = pltpu.async_copy(utab_hbm.at[uidx_v], urows_v, sem)
    cp = pltpu.async_copy(itab_hbm.at[pidx_v], prows_v, sem)
    ncopies = [
        pltpu.async_copy(itab_hbm.at[nidx_v.at[j]],
                         nrows_v.at[pl.ds(j * BPW, BPW)], nsems[j])
        for j in range(K)
    ]
    cu.wait()
    cp.wait()

    zero = jnp.zeros((LANES,), jnp.float32)
    nchunks = DIM // LANES
    iota = lax.iota(jnp.int32, LANES)
    lane15 = iota == 15
    def row_step(b, carry):
      su, sp, sn = carry
      uc = [urows_v[b, pl.ds(c * LANES, LANES)] for c in range(nchunks)]
      pc = [prows_v[b, pl.ds(c * LANES, LANES)] for c in range(nchunks)]
      for c in range(nchunks):
        su = su + uc[c] * uc[c]
        sp = sp + pc[c] * pc[c]
      for k in range(K):
        nb = b * K + k
        nc = [nrows_v[nb, pl.ds(c * LANES, LANES)] for c in range(nchunks)]
        for c in range(nchunks):
          sn = sn + nc[c] * nc[c]
        wd = uc[0] * (pc[0] - nc[0])
        for c in range(1, nchunks):
          wd = wd + uc[c] * (pc[c] - nc[c])
        plsc.store_scatter(pn_v, [iota + (nb - 15)], plsc.cumsum(wd),
                           mask=lane15)
      return su, sp, sn

    s_u, s_p, s_n = zero, zero, zero
    rows_per_chunk = BPW // K
    for j in range(K):
      ncopies[j].wait()
      s_u, s_p, s_n = lax.fori_loop(j * rows_per_chunk,
                                    (j + 1) * rows_per_chunk,
                                    row_step, (s_u, s_p, s_n))

    nrm_v[pl.ds(0, LANES)] = s_u
    nrm_v[pl.ds(LANES, LANES)] = s_p
    nrm_v[pl.ds(2 * LANES, LANES)] = s_n * (1.0 / K)

    pltpu.sync_copy(pn_v, pn_hbm.at[pl.ds(base * K, BPW * K)])
    pltpu.sync_copy(nrm_v, norms_hbm.at[pl.ds(wid * 48, 48)])

  return _sc_gather_score


def _tc_loss_body(pn_ref, nrm_ref, mf_ref, emb_ref, tot_ref):
  x = -pn_ref[...]
  sp = jnp.maximum(x, 0.0) + jnp.log1p(jnp.exp(-jnp.abs(x)))
  mf = jnp.sum(sp) * (1.0 / (BATCH * K))
  reg = jnp.sum(nrm_ref[...]) * 0.5
  emb = (DECAY / BATCH) * reg
  one = jnp.ones((1, 1), jnp.float32)
  mf_ref[...] = mf * one
  emb_ref[...] = emb * one
  tot_ref[...] = (mf + emb) * one


def kernel(user_table, item_table, users, pos_items, neg_items):
  users_i = users.astype(jnp.int32)
  pos_i = pos_items.astype(jnp.int32)
  neg_i = neg_items.astype(jnp.int32).reshape(NUM_WORKERS, K, BPW)

  pn_flat, norms = _make_sc_kernel()(users_i, pos_i, neg_i,
                                     user_table, item_table)
  pos_neg = pn_flat.reshape(BATCH, K)

  mf, emb, tot = pl.pallas_call(
      _tc_loss_body,
      out_shape=[jax.ShapeDtypeStruct((1, 1), jnp.float32)] * 3,
  )(pn_flat, norms)

  return (tot.reshape(()), mf.reshape(()), emb.reshape(()), pos_neg)


# speedup vs baseline: 1.7318x; 1.0132x over previous
import functools

import jax
import jax.numpy as jnp
from jax import lax
from jax.experimental import pallas as pl
from jax.experimental.pallas import tpu as pltpu
from jax.experimental.pallas import tpu_sc as plsc

N_USERS = 100000
M_ITEMS = 100000
DIM = 64
BATCH = 4096
K = 8
DECAY = 0.0001

NUM_WORKERS = 32
BPW = BATCH // NUM_WORKERS
LANES = 16

@functools.cache
def _make_sc_kernel():
  mesh = plsc.VectorSubcoreMesh(core_axis_name="c", subcore_axis_name="s")

  @functools.partial(
      pl.kernel,
      mesh=mesh,
      compiler_params=pltpu.CompilerParams(needs_layout_passes=False,
                                           use_tc_tiling_on_sc=False),
      out_type=[
          jax.ShapeDtypeStruct((BATCH * K,), jnp.float32),
          jax.ShapeDtypeStruct((NUM_WORKERS * 48,), jnp.float32),
      ],
      scratch_types=[
          pltpu.VMEM((BPW,), jnp.int32),
          pltpu.VMEM((BPW,), jnp.int32),
          pltpu.VMEM((K, BPW), jnp.int32),
          pltpu.VMEM((BPW, DIM), jnp.float32),
          pltpu.VMEM((BPW, DIM), jnp.float32),
          pltpu.VMEM((BPW * K, DIM), jnp.float32),
          pltpu.VMEM((BPW * K,), jnp.float32),
          pltpu.VMEM((48,), jnp.float32),
          pltpu.SemaphoreType.DMA,
      ],
  )
  def _sc_gather_score(users_hbm, pos_hbm, neg_hbm, utab_hbm, itab_hbm,
                       pn_hbm, norms_hbm,
                       uidx_v, pidx_v, nidx_v, urows_v, prows_v, nrows_v,
                       pn_v, nrm_v, sem):
    wid = lax.axis_index("s") * 2 + lax.axis_index("c")
    base = wid * BPW

    pltpu.sync_copy(users_hbm.at[pl.ds(base, BPW)], uidx_v)
    pltpu.sync_copy(pos_hbm.at[pl.ds(base, BPW)], pidx_v)
    pltpu.sync_copy(neg_hbm.at[wid], nidx_v)

    copies = [
        pltpu.async_copy(utab_hbm.at[uidx_v], urows_v, sem),
        pltpu.async_copy(itab_hbm.at[pidx_v], prows_v, sem),
    ]
    for j in range(K):
      copies.append(
          pltpu.async_copy(itab_hbm.at[nidx_v.at[j]],
                           nrows_v.at[pl.ds(j * BPW, BPW)], sem))
    for c in copies:
      c.wait()

    zero = jnp.zeros((LANES,), jnp.float32)
    nchunks = DIM // LANES
    iota = lax.iota(jnp.int32, LANES)
    lane15 = iota == 15
    def row_step(b, carry):
      su, sp, sn = carry
      uc = [urows_v[b, pl.ds(c * LANES, LANES)] for c in range(nchunks)]
      pc = [prows_v[b, pl.ds(c * LANES, LANES)] for c in range(nchunks)]
      for c in range(nchunks):
        su = su + uc[c] * uc[c]
        sp = sp + pc[c] * pc[c]
      for k in range(K):
        nb = b * K + k
        nc = [nrows_v[nb, pl.ds(c * LANES, LANES)] for c in range(nchunks)]
        for c in range(nchunks):
          sn = sn + nc[c] * nc[c]
        wd = uc[0] * (pc[0] - nc[0])
        for c in range(1, nchunks):
          wd = wd + uc[c] * (pc[c] - nc[c])
        plsc.store_scatter(pn_v, [iota + (nb - 15)], plsc.cumsum(wd),
                           mask=lane15)
      return su, sp, sn

    s_u, s_p, s_n = lax.fori_loop(0, BPW, row_step, (zero, zero, zero))

    nrm_v[pl.ds(0, LANES)] = s_u
    nrm_v[pl.ds(LANES, LANES)] = s_p
    nrm_v[pl.ds(2 * LANES, LANES)] = s_n * (1.0 / K)

    pltpu.sync_copy(pn_v, pn_hbm.at[pl.ds(base * K, BPW * K)])
    pltpu.sync_copy(nrm_v, norms_hbm.at[pl.ds(wid * 48, 48)])

  return _sc_gather_score


def _tc_loss_body(pn_ref, nrm_ref, mf_ref, emb_ref, tot_ref):
  x = -pn_ref[...]
  sp = jnp.maximum(x, 0.0) + jnp.log1p(jnp.exp(-jnp.abs(x)))
  mf = jnp.sum(sp) * (1.0 / (BATCH * K))
  reg = jnp.sum(nrm_ref[...]) * 0.5
  emb = (DECAY / BATCH) * reg
  one = jnp.ones((1, 1), jnp.float32)
  mf_ref[...] = mf * one
  emb_ref[...] = emb * one
  tot_ref[...] = (mf + emb) * one


def kernel(user_table, item_table, users, pos_items, neg_items):
  users_i = users.astype(jnp.int32)
  pos_i = pos_items.astype(jnp.int32)
  neg_i = neg_items.astype(jnp.int32).reshape(NUM_WORKERS, K, BPW)

  pn_flat, norms = _make_sc_kernel()(users_i, pos_i, neg_i,
                                     user_table, item_table)
  pos_neg = pn_flat.reshape(BATCH, K)

  mf, emb, tot = pl.pallas_call(
      _tc_loss_body,
      out_shape=[jax.ShapeDtypeStruct((1, 1), jnp.float32)] * 3,
  )(pn_flat, norms)

  return (tot.reshape(()), mf.reshape(()), emb.reshape(()), pos_neg)
